# Initial kernel scaffold; baseline (speedup 1.0000x reference)
import functools

import jax
import jax.numpy as jnp
from jax import lax
from jax.experimental import pallas as pl
from jax.experimental.pallas import tpu as pltpu
from jax.experimental.pallas import tpu_sc as plsc

_NUM_FIELDS = 26
_VOCAB = 100000
_EMB_DIM = 32
_BATCH = 16384

_NC = 2
_NS = 16
_NW = _NC * _NS
_LANES = 16

_ROWS = _NUM_FIELDS * _BATCH
_BLK = 128
_NBLK = 8
_CHUNK = _BLK * _NBLK
_NCHUNK = _ROWS // _CHUNK
_CPW = _NCHUNK // _NW
_CHUNKS_PER_FIELD = _BATCH // _CHUNK


@functools.partial(
    pl.kernel,
    out_type=jax.ShapeDtypeStruct((_NCHUNK, _NBLK, _BLK, _EMB_DIM), jnp.float32),
    mesh=plsc.VectorSubcoreMesh(core_axis_name="c", subcore_axis_name="s"),
    scratch_types=[
        pltpu.VMEM((_NBLK, _BLK), jnp.int32),
        pltpu.VMEM((_NBLK, _BLK, _EMB_DIM), jnp.float32),
        pltpu.SemaphoreType.DMA,
    ],
)
def _mce_gather(idx_hbm, tab_hbm, out_hbm, idx_v, rows_v, sem):
    w = lax.axis_index("s") * _NC + lax.axis_index("c")
    c0 = w * _CPW

    def chunk_body(i, carry):
        c = c0 + i
        pltpu.sync_copy(idx_hbm.at[c], idx_v)
        off = (c // _CHUNKS_PER_FIELD) * _VOCAB
        for j in range(_NBLK):
            for t in range(_BLK // _LANES):
                sl = pl.ds(t * _LANES, _LANES)
                idx_v[j, sl] = idx_v[j, sl] + off
        copies = [
            pltpu.async_copy(tab_hbm.at[idx_v.at[j]], rows_v.at[j], sem)
            for j in range(_NBLK)
        ]
        for cp in copies:
            cp.wait()
        pltpu.sync_copy(rows_v, out_hbm.at[c])
        return carry

    lax.fori_loop(0, _CPW, chunk_body, 0)


def kernel(inputs, tables):
    idx = inputs.astype(jnp.int32).T.reshape(_NCHUNK, _NBLK, _BLK)
    tab = tables.reshape(_NUM_FIELDS * _VOCAB, _EMB_DIM)
    out = _mce_gather(idx, tab)
    return out.reshape(_NUM_FIELDS, _BATCH, 1, _EMB_DIM)


# speedup vs baseline: 1.1564x; 1.1564x over previous
import functools

import jax
import jax.numpy as jnp
from jax import lax
from jax.experimental import pallas as pl
from jax.experimental.pallas import tpu as pltpu
from jax.experimental.pallas import tpu_sc as plsc

_NUM_FIELDS = 26
_VOCAB = 100000
_EMB_DIM = 32
_BATCH = 16384

_NC = 2
_NS = 16
_NW = _NC * _NS
_LANES = 16

_ROWS = _NUM_FIELDS * _BATCH
_BLK = 128
_NBLK = 8
_CHUNK = _BLK * _NBLK
_NCHUNK = _ROWS // _CHUNK
_CPW = _NCHUNK // _NW
_CHUNKS_PER_FIELD = _BATCH // _CHUNK


@functools.partial(
    pl.kernel,
    out_type=jax.ShapeDtypeStruct((_NCHUNK, _NBLK, _BLK, _EMB_DIM), jnp.float32),
    mesh=plsc.VectorSubcoreMesh(core_axis_name="c", subcore_axis_name="s"),
    compiler_params=pltpu.CompilerParams(use_tc_tiling_on_sc=False),
    scratch_types=[
        pltpu.VMEM((_NBLK, _BLK), jnp.int32),
        pltpu.VMEM((_NBLK, _BLK, _EMB_DIM), jnp.float32),
        pltpu.SemaphoreType.DMA,
    ],
)
def _mce_gather(idx_hbm, tab_hbm, out_hbm, idx_v, rows_v, sem):
    w = lax.axis_index("s") * _NC + lax.axis_index("c")
    c0 = w * _CPW

    def chunk_body(i, carry):
        c = c0 + i
        pltpu.sync_copy(idx_hbm.at[c], idx_v)
        off = (c // _CHUNKS_PER_FIELD) * _VOCAB
        for j in range(_NBLK):
            for t in range(_BLK // _LANES):
                sl = pl.ds(t * _LANES, _LANES)
                idx_v[j, sl] = idx_v[j, sl] + off
        copies = [
            pltpu.async_copy(tab_hbm.at[idx_v.at[j]], rows_v.at[j], sem)
            for j in range(_NBLK)
        ]
        for cp in copies:
            cp.wait()
        pltpu.sync_copy(rows_v, out_hbm.at[c])
        return carry

    lax.fori_loop(0, _CPW, chunk_body, 0)


def kernel(inputs, tables):
    idx = inputs.astype(jnp.int32).T.reshape(_NCHUNK, _NBLK, _BLK)
    tab = tables.reshape(_NUM_FIELDS * _VOCAB, _EMB_DIM)
    out = _mce_gather(idx, tab)
    return out.reshape(_NUM_FIELDS, _BATCH, 1, _EMB_DIM)
